# baseline (speedup 1.0000x reference)
import functools

import jax
import jax.numpy as jnp
from jax import lax
from jax.experimental import pallas as pl
from jax.experimental.pallas import tpu as pltpu
from jax.experimental.pallas import tpu_sc as plsc

N = 10000
E = 320000
D = 128
LYR = 5
NAF = 9
AV = 128
NBF = 3
BV = 16

NC = 2
NS = 16
NW = NC * NS
CH = 128
NCHUNK = 80
EPW = CH * NCHUNK
EPAD = EPW * NW
NPAD = 10240
RPT = NPAD // NS

@functools.lru_cache(maxsize=None)
def _mesh():
    return plsc.VectorSubcoreMesh(core_axis_name="c", subcore_axis_name="s",
                                  num_cores=NC, num_subcores=NS)


def _deg_body(rowd3, zrow, out, rowd_v, ones_v, acc):
    c = lax.axis_index("c")
    s = lax.axis_index("s")
    w = c * NS + s
    pltpu.sync_copy(zrow, acc.at[pl.ds(s * RPT, RPT)])
    for k in range(CH // 16):
        ones_v[pl.ds(k * 16, 16)] = jnp.ones((16,), jnp.float32)
    plsc.subcore_barrier()
    pltpu.sync_copy(rowd3.at[w], rowd_v)

    def chunk(j, carry):
        pltpu.sync_copy(ones_v, acc.at[rowd_v.at[j]], add=True)
        return carry

    lax.fori_loop(0, NCHUNK, chunk, 0)
    plsc.subcore_barrier()
    pltpu.sync_copy(acc.at[pl.ds(s * RPT, RPT)], out.at[c, pl.ds(s * RPT, RPT)])


@functools.lru_cache(maxsize=None)
def _deg_sc():
    return pl.kernel(
        _deg_body,
        out_type=jax.ShapeDtypeStruct((NC, NPAD), jnp.float32),
        mesh=_mesh(),
        compiler_params=pltpu.CompilerParams(needs_layout_passes=False),
        scratch_types=[
            pltpu.VMEM((NCHUNK, CH), jnp.int32),
            pltpu.VMEM((CH,), jnp.float32),
            pltpu.VMEM_SHARED((NPAD,), jnp.float32),
        ],
    )


CHL = 32
NCHT = EPAD // CHL
K0 = 480
K1 = NCHT // NS - K0
NSL = 4


def _layer_body(xl, ee, dinv, idx4, out, *scr):
    stg = scr[0:4]
    colb = scr[4:8]
    xb = scr[8:12]
    eb = scr[12:16]
    dinv_v, nrbuf, agg = scr[16], scr[17], scr[18]
    si = scr[19:23]
    sg = scr[23:27]
    ss = scr[27:31]

    c = lax.axis_index("c")
    s = lax.axis_index("s")

    def zf(i, c2):
        for k in range(D // 16):
            xb[0][i, pl.ds(k * 16, 16)] = jnp.zeros((16,), jnp.float32)
        return c2

    lax.fori_loop(0, CHL, zf, 0)
    for t in range(RPT // CHL):
        pltpu.sync_copy(xb[0], agg.at[pl.ds(s * RPT + t * CHL, CHL)])
    pltpu.sync_copy(dinv, dinv_v)
    plsc.subcore_barrier()

    def issue_g(b):
        pltpu.async_copy(xl.at[stg[b].at[0]], xb[b], sg[b])
        pltpu.async_copy(ee.at[stg[b].at[2]], eb[b], sg[b])

    def wait_g(b):
        pltpu.make_async_copy(xl.at[stg[b].at[0]], xb[b], sg[b]).wait()
        pltpu.make_async_copy(ee.at[stg[b].at[2]], eb[b], sg[b]).wait()

    def issue_s(b):
        pltpu.async_copy(xb[b], agg.at[colb[b]], ss[b], add=True)

    def wait_s(b):
        pltpu.make_async_copy(xb[b], agg.at[colb[b]], ss[b]).wait()

    def issue_i(base, j, b):
        pltpu.async_copy(idx4.at[base + j], stg[b], si[b])

    def wait_i(b):
        pltpu.make_async_copy(idx4.at[0], stg[b], si[b]).wait()

    def compute(b):
        stg_b, colb_b, xb_b, eb_b = stg[b], colb[b], xb[b], eb[b]

        def grp(g, c2):
            sl = pl.ds(g * 16, 16)
            nrbuf[sl] = plsc.load_gather(dinv_v, [stg_b[0, sl]])
            colb_b[sl] = stg_b[1, sl]
            return c2

        lax.fori_loop(0, CHL // 16, grp, 0)

        def edge(i, c2):
            nr = nrbuf[pl.ds(i, 16)][0]
            for k in range(D // 16):
                sl = pl.ds(k * 16, 16)
                xb_b[i, sl] = jnp.maximum(xb_b[i, sl] + eb_b[i, sl], 0.0) * nr
            return c2

        lax.fori_loop(0, CHL, edge, 0)

    def step(base, j, b, do_swait, do_g2, do_iissue):
        b2 = (b + 2) % NSL
        if do_g2:
            wait_i(b2)
            if do_swait:
                wait_s(b2)
            issue_g(b2)
        wait_g(b)
        compute(b)
        issue_s(b)
        if do_iissue:
            issue_i(base, j + NSL, b)

    def pipeline(base, kt):
        pltpu.sync_copy(idx4.at[base], stg[0])
        pltpu.sync_copy(idx4.at[base + 1], stg[1])
        issue_g(0)
        issue_g(1)
        issue_i(base, 2, 2)
        issue_i(base, 3, 3)

        step(base, 0, 0, False, True, True)
        step(base, 1, 1, False, True, True)
        step(base, 2, 2, True, True, True)
        step(base, 3, 3, True, True, True)

        def outer(m, carry):
            for b in range(NSL):
                step(base, NSL * m + b, b, True, True, True)
            return carry

        lax.fori_loop(1, kt // NSL - 1, outer, 0)

        jl = kt - NSL
        step(base, jl + 0, 0, True, True, False)
        step(base, jl + 1, 1, True, True, False)
        step(base, jl + 2, 2, False, False, False)
        step(base, jl + 3, 3, False, False, False)

        for b in range(NSL):
            wait_s(b)

    @pl.when(c == 0)
    def _run0():
        pipeline(s * K0, K0)

    @pl.when(c == 1)
    def _run1():
        pipeline(NS * K0 + s * K1, K1)

    plsc.subcore_barrier()
    pltpu.sync_copy(agg.at[pl.ds(s * RPT, RPT)],
                    out.at[c, pl.ds(s * RPT, RPT)])


@functools.lru_cache(maxsize=None)
def _layer_sc():
    return pl.kernel(
        _layer_body,
        out_type=jax.ShapeDtypeStruct((NC, NPAD, D), jnp.float32),
        mesh=_mesh(),
        compiler_params=pltpu.CompilerParams(needs_layout_passes=False),
        scratch_types=(
            [pltpu.VMEM((3, CHL), jnp.int32)] * NSL
            + [pltpu.VMEM((CHL,), jnp.int32)] * NSL
            + [pltpu.VMEM((CHL, D), jnp.float32)] * NSL
            + [pltpu.VMEM((CHL, D), jnp.float32)] * NSL
            + [pltpu.VMEM((NPAD,), jnp.float32),
               pltpu.VMEM((CHL + 16,), jnp.float32),
               pltpu.VMEM_SHARED((NPAD, D), jnp.float32)]
            + [pltpu.SemaphoreType.DMA] * (3 * NSL)
        ),
    )


NBLK = 10
BN_ROWS = N // NBLK


def _embed_body(x_ref, at_ref, w0_ref, b0_ref, h0_ref, xl0_ref):
    xv = x_ref[...]
    iot = lax.broadcasted_iota(jnp.int32, (1, AV), 1)
    h = jnp.zeros((BN_ROWS, D), jnp.float32)
    for k in range(NAF):
        oh = (xv[:, k:k + 1] == iot).astype(jnp.float32)
        h = h + jnp.dot(oh, at_ref[k], preferred_element_type=jnp.float32,
                        precision=lax.Precision.HIGHEST)
    h0_ref[...] = h
    xl0_ref[...] = jnp.dot(h, w0_ref[...],
                           preferred_element_type=jnp.float32) + b0_ref[...]


_embed_tc = pl.pallas_call(
    _embed_body,
    grid=(NBLK,),
    in_specs=[
        pl.BlockSpec((BN_ROWS, NAF), lambda i: (i, 0)),
        pl.BlockSpec((NAF, AV, D), lambda i: (0, 0, 0)),
        pl.BlockSpec((D, D), lambda i: (0, 0)),
        pl.BlockSpec((1, D), lambda i: (0, 0)),
    ],
    out_specs=[
        pl.BlockSpec((BN_ROWS, D), lambda i: (i, 0)),
        pl.BlockSpec((BN_ROWS, D), lambda i: (i, 0)),
    ],
    out_shape=[
        jax.ShapeDtypeStruct((N, D), jnp.float32),
        jax.ShapeDtypeStruct((N, D), jnp.float32),
    ],
)


def _misc_body(degp_ref, bt_ref, dinv_ref, deginv_ref, ee_ref):
    degp = degp_ref[...]
    deg = degp[0, :N] + degp[1, :N] + 1.0
    dinv = lax.rsqrt(deg)[:, None]
    deginv = (1.0 / deg)[:, None]
    pad = jnp.zeros((NPAD - N, 1), jnp.float32)
    dinv_ref[...] = jnp.concatenate([dinv, pad], axis=0)
    deginv_ref[...] = deginv

    bt = bt_ref[...]
    for l in range(LYR):
        a = jnp.broadcast_to(bt[l, 0][:, None, :], (BV, BV * BV, D))
        a = a.reshape(BV * BV * BV, D)
        bb = jnp.broadcast_to(bt[l, 1][:, None, :], (BV, BV, D))
        bb = bb.reshape(BV * BV, D)
        bb = jnp.broadcast_to(bb[None, :, :], (BV, BV * BV, D))
        bb = bb.reshape(BV * BV * BV, D)
        cc = jnp.broadcast_to(bt[l, 2][None, :, :], (BV * BV, BV, D))
        cc = cc.reshape(BV * BV * BV, D)
        ee_ref[l] = a + bb + cc


_misc_tc = pl.pallas_call(
    _misc_body,
    out_shape=[
        jax.ShapeDtypeStruct((NPAD, 1), jnp.float32),
        jax.ShapeDtypeStruct((N, 1), jnp.float32),
        jax.ShapeDtypeStruct((LYR, BV * BV * BV, D), jnp.float32),
    ],
)


def _epi_body(last, s_ref, xl_ref, dinv_ref, deginv_ref, root_ref, g_ref,
              be_ref, *rest):
    if last:
        (h_ref,) = rest
    else:
        wn_ref, bn_ref, h_ref, xln_ref = rest
    sp = s_ref[...]
    t = ((sp[0] + sp[1]) * dinv_ref[...]
         + jax.nn.relu(xl_ref[...] + root_ref[...]) * deginv_ref[...])
    m = jnp.mean(t, axis=0, keepdims=True)
    v = jnp.mean((t - m) ** 2, axis=0, keepdims=True)
    hb = g_ref[...] * (t - m) * lax.rsqrt(v + 1e-5) + be_ref[...]
    if not last:
        hb = jax.nn.relu(hb)
    h_ref[...] = hb
    if not last:
        xln_ref[...] = jnp.dot(hb, wn_ref[...],
                               preferred_element_type=jnp.float32) + bn_ref[...]


_TC_PARAMS = pltpu.CompilerParams(vmem_limit_bytes=100 * 1024 * 1024)

_epi_tc = pl.pallas_call(
    functools.partial(_epi_body, False),
    out_shape=[
        jax.ShapeDtypeStruct((N, D), jnp.float32),
        jax.ShapeDtypeStruct((N, D), jnp.float32),
    ],
    compiler_params=_TC_PARAMS,
)

_epi_tc_last = pl.pallas_call(
    functools.partial(_epi_body, True),
    out_shape=[jax.ShapeDtypeStruct((N, D), jnp.float32)],
    compiler_params=_TC_PARAMS,
)


def kernel(x, edge_index, edge_attr, batch, atom_tab, bond_tab, W, b, root,
           gamma, beta):
    row = edge_index[0].astype(jnp.int32)
    col = edge_index[1].astype(jnp.int32)
    ea = edge_attr.astype(jnp.int32)
    aidx = ea[:, 0] * (BV * BV) + ea[:, 1] * BV + ea[:, 2]

    pad_e = EPAD - E
    row_p = jnp.concatenate([row, jnp.zeros((pad_e,), jnp.int32)])
    rowd_p = jnp.concatenate([row, jnp.full((pad_e,), N, jnp.int32)])
    col_p = jnp.concatenate([col, jnp.full((pad_e,), N, jnp.int32)])
    aidx_p = jnp.concatenate([aidx, jnp.zeros((pad_e,), jnp.int32)])
    rowd3 = rowd_p.reshape(NW, NCHUNK, CH)
    idx4 = jnp.stack([row_p.reshape(NCHT, CHL),
                      col_p.reshape(NCHT, CHL),
                      aidx_p.reshape(NCHT, CHL)], axis=1)

    zrow = jnp.zeros((RPT,), jnp.float32)

    degp = _deg_sc()(rowd3, zrow)

    b2 = b.reshape(LYR, 1, D)
    root2 = root.reshape(LYR, 1, D)
    gamma2 = gamma.reshape(LYR, 1, D)
    beta2 = beta.reshape(LYR, 1, D)

    h0, xl = _embed_tc(x.astype(jnp.int32), atom_tab, W[0], b2[0])
    dinv_col, deginv_col, ee = _misc_tc(degp, bond_tab)
    dinv_flat = dinv_col.reshape(NPAD)

    hs = [h0]
    for l in range(LYR):
        aggout = _layer_sc()(xl, ee[l], dinv_flat, idx4)
        sp = aggout[:, :N, :]
        if l < LYR - 1:
            h, xl = _epi_tc(sp, xl, dinv_col[:N], deginv_col, root2[l],
                            gamma2[l], beta2[l], W[l + 1], b2[l + 1])
        else:
            (h,) = _epi_tc_last(sp, xl, dinv_col[:N], deginv_col, root2[l],
                                gamma2[l], beta2[l])
        hs.append(h)

    return hs[-1], jnp.stack(hs, axis=0)


# speedup vs baseline: 1.0546x; 1.0546x over previous
import functools

import jax
import jax.numpy as jnp
from jax import lax
from jax.experimental import pallas as pl
from jax.experimental.pallas import tpu as pltpu
from jax.experimental.pallas import tpu_sc as plsc

N = 10000
E = 320000
D = 128
LYR = 5
NAF = 9
AV = 128
NBF = 3
BV = 16

NC = 2
NS = 16
NW = NC * NS
CH = 128
NCHUNK = 80
EPW = CH * NCHUNK
EPAD = EPW * NW
NPAD = 10240
RPT = NPAD // NS

@functools.lru_cache(maxsize=None)
def _mesh():
    return plsc.VectorSubcoreMesh(core_axis_name="c", subcore_axis_name="s",
                                  num_cores=NC, num_subcores=NS)


def _deg_body(rowd3, zrow, out, rowd_v, ones_v, acc):
    c = lax.axis_index("c")
    s = lax.axis_index("s")
    w = c * NS + s
    pltpu.sync_copy(zrow, acc.at[pl.ds(s * RPT, RPT)])
    for k in range(CH // 16):
        ones_v[pl.ds(k * 16, 16)] = jnp.ones((16,), jnp.float32)
    plsc.subcore_barrier()
    pltpu.sync_copy(rowd3.at[w], rowd_v)

    def chunk(j, carry):
        pltpu.sync_copy(ones_v, acc.at[rowd_v.at[j]], add=True)
        return carry

    lax.fori_loop(0, NCHUNK, chunk, 0)
    plsc.subcore_barrier()
    pltpu.sync_copy(acc.at[pl.ds(s * RPT, RPT)], out.at[c, pl.ds(s * RPT, RPT)])


@functools.lru_cache(maxsize=None)
def _deg_sc():
    return pl.kernel(
        _deg_body,
        out_type=jax.ShapeDtypeStruct((NC, NPAD), jnp.float32),
        mesh=_mesh(),
        compiler_params=pltpu.CompilerParams(needs_layout_passes=False),
        scratch_types=[
            pltpu.VMEM((NCHUNK, CH), jnp.int32),
            pltpu.VMEM((CH,), jnp.float32),
            pltpu.VMEM_SHARED((NPAD,), jnp.float32),
        ],
    )


CHL = 64
NCHL = EPW // CHL
NCHT = EPAD // CHL
K0 = 256
K1 = NCHT // NS - K0


def _layer_body(xl, ee, dinv, idx4, out, *scr):
    stg = scr[0:2]
    colb = scr[2:4]
    xb = scr[4:6]
    eb = scr[6:8]
    dinv_v, nrbuf, agg = scr[8], scr[9], scr[10]
    si = scr[11:13]
    sg = scr[13:15]
    ss = scr[15:17]

    c = lax.axis_index("c")
    s = lax.axis_index("s")

    def zf(i, c2):
        for k in range(D // 16):
            xb[0][i, pl.ds(k * 16, 16)] = jnp.zeros((16,), jnp.float32)
        return c2

    lax.fori_loop(0, CHL, zf, 0)
    for t in range(RPT // CHL):
        pltpu.sync_copy(xb[0], agg.at[pl.ds(s * RPT + t * CHL, CHL)])
    pltpu.sync_copy(dinv, dinv_v)
    plsc.subcore_barrier()

    def issue_g(b):
        pltpu.async_copy(xl.at[stg[b].at[0]], xb[b], sg[b])
        pltpu.async_copy(ee.at[stg[b].at[2]], eb[b], sg[b])

    def wait_g(b):
        pltpu.make_async_copy(xl.at[stg[b].at[0]], xb[b], sg[b]).wait()
        pltpu.make_async_copy(ee.at[stg[b].at[2]], eb[b], sg[b]).wait()

    def issue_s(b):
        pltpu.async_copy(xb[b], agg.at[colb[b]], ss[b], add=True)

    def wait_s(b):
        pltpu.make_async_copy(xb[b], agg.at[colb[b]], ss[b]).wait()

    def issue_i(base, j, b):
        pltpu.async_copy(idx4.at[base + j], stg[b], si[b])

    def wait_i(b):
        pltpu.make_async_copy(idx4.at[0], stg[b], si[b]).wait()

    def compute(b):
        stg_b, colb_b, xb_b, eb_b = stg[b], colb[b], xb[b], eb[b]

        def grp(g, c2):
            sl = pl.ds(g * 16, 16)
            nrbuf[sl] = plsc.load_gather(dinv_v, [stg_b[0, sl]])
            colb_b[sl] = stg_b[1, sl]
            return c2

        lax.fori_loop(0, CHL // 16, grp, 0)

        def edge(i, c2):
            nr = nrbuf[pl.ds(i, 16)][0]
            for k in range(D // 16):
                sl = pl.ds(k * 16, 16)
                xb_b[i, sl] = jnp.maximum(xb_b[i, sl] + eb_b[i, sl], 0.0) * nr
            return c2

        lax.fori_loop(0, CHL, edge, 0)

    def step(base, j, b, do_swait, do_g1, do_iissue):
        o = 1 - b
        if do_g1:
            wait_i(o)
            if do_swait:
                wait_s(o)
            issue_g(o)
        wait_g(b)
        compute(b)
        issue_s(b)
        if do_iissue:
            issue_i(base, j + 2, b)

    def pipeline(base, kt):
        pltpu.sync_copy(idx4.at[base], stg[0])
        issue_g(0)
        issue_i(base, 1, 1)

        step(base, 0, 0, False, True, True)

        def outer(m, carry):
            step(base, 2 * m + 1, 1, True, True, True)
            step(base, 2 * m + 2, 0, True, True, True)
            return carry

        lax.fori_loop(0, (kt - 4) // 2, outer, 0)

        step(base, kt - 3, 1, True, True, True)
        step(base, kt - 2, 0, True, True, False)
        step(base, kt - 1, 1, True, False, False)

        wait_s(0)
        wait_s(1)

    @pl.when(c == 0)
    def _run0():
        pipeline(s * K0, K0)

    @pl.when(c == 1)
    def _run1():
        pipeline(NS * K0 + s * K1, K1)

    plsc.subcore_barrier()
    pltpu.sync_copy(agg.at[pl.ds(s * RPT, RPT)],
                    out.at[c, pl.ds(s * RPT, RPT)])


@functools.lru_cache(maxsize=None)
def _layer_sc():
    return pl.kernel(
        _layer_body,
        out_type=jax.ShapeDtypeStruct((NC, NPAD, D), jnp.float32),
        mesh=_mesh(),
        compiler_params=pltpu.CompilerParams(needs_layout_passes=False),
        scratch_types=(
            [pltpu.VMEM((3, CHL), jnp.int32)] * 2
            + [pltpu.VMEM((CHL,), jnp.int32)] * 2
            + [pltpu.VMEM((CHL, D), jnp.float32)] * 2
            + [pltpu.VMEM((CHL, D), jnp.float32)] * 2
            + [pltpu.VMEM((NPAD,), jnp.float32),
               pltpu.VMEM((CHL + 16,), jnp.float32),
               pltpu.VMEM_SHARED((NPAD, D), jnp.float32)]
            + [pltpu.SemaphoreType.DMA] * 6
        ),
    )


NBLK = 10
BN_ROWS = N // NBLK


def _embed_body(x_ref, at_ref, w0_ref, b0_ref, h0_ref, xl0_ref):
    xv = x_ref[...]
    iot = lax.broadcasted_iota(jnp.int32, (1, AV), 1)
    h = jnp.zeros((BN_ROWS, D), jnp.float32)
    for k in range(NAF):
        oh = (xv[:, k:k + 1] == iot).astype(jnp.float32)
        h = h + jnp.dot(oh, at_ref[k], preferred_element_type=jnp.float32,
                        precision=lax.Precision.HIGHEST)
    h0_ref[...] = h
    xl0_ref[...] = jnp.dot(h, w0_ref[...],
                           preferred_element_type=jnp.float32) + b0_ref[...]


_embed_tc = pl.pallas_call(
    _embed_body,
    grid=(NBLK,),
    in_specs=[
        pl.BlockSpec((BN_ROWS, NAF), lambda i: (i, 0)),
        pl.BlockSpec((NAF, AV, D), lambda i: (0, 0, 0)),
        pl.BlockSpec((D, D), lambda i: (0, 0)),
        pl.BlockSpec((1, D), lambda i: (0, 0)),
    ],
    out_specs=[
        pl.BlockSpec((BN_ROWS, D), lambda i: (i, 0)),
        pl.BlockSpec((BN_ROWS, D), lambda i: (i, 0)),
    ],
    out_shape=[
        jax.ShapeDtypeStruct((N, D), jnp.float32),
        jax.ShapeDtypeStruct((N, D), jnp.float32),
    ],
)


def _misc_body(degp_ref, bt_ref, dinv_ref, deginv_ref, ee_ref):
    degp = degp_ref[...]
    deg = degp[0, :N] + degp[1, :N] + 1.0
    dinv = lax.rsqrt(deg)[:, None]
    deginv = (1.0 / deg)[:, None]
    pad = jnp.zeros((NPAD - N, 1), jnp.float32)
    dinv_ref[...] = jnp.concatenate([dinv, pad], axis=0)
    deginv_ref[...] = deginv

    bt = bt_ref[...]
    for l in range(LYR):
        a = jnp.broadcast_to(bt[l, 0][:, None, :], (BV, BV * BV, D))
        a = a.reshape(BV * BV * BV, D)
        bb = jnp.broadcast_to(bt[l, 1][:, None, :], (BV, BV, D))
        bb = bb.reshape(BV * BV, D)
        bb = jnp.broadcast_to(bb[None, :, :], (BV, BV * BV, D))
        bb = bb.reshape(BV * BV * BV, D)
        cc = jnp.broadcast_to(bt[l, 2][None, :, :], (BV * BV, BV, D))
        cc = cc.reshape(BV * BV * BV, D)
        ee_ref[l] = a + bb + cc


_misc_tc = pl.pallas_call(
    _misc_body,
    out_shape=[
        jax.ShapeDtypeStruct((NPAD, 1), jnp.float32),
        jax.ShapeDtypeStruct((N, 1), jnp.float32),
        jax.ShapeDtypeStruct((LYR, BV * BV * BV, D), jnp.float32),
    ],
)


def _epi_body(last, s_ref, xl_ref, dinv_ref, deginv_ref, root_ref, g_ref,
              be_ref, *rest):
    if last:
        (h_ref,) = rest
    else:
        wn_ref, bn_ref, h_ref, xln_ref = rest
    sp = s_ref[...]
    t = ((sp[0] + sp[1]) * dinv_ref[...]
         + jax.nn.relu(xl_ref[...] + root_ref[...]) * deginv_ref[...])
    m = jnp.mean(t, axis=0, keepdims=True)
    v = jnp.mean((t - m) ** 2, axis=0, keepdims=True)
    hb = g_ref[...] * (t - m) * lax.rsqrt(v + 1e-5) + be_ref[...]
    if not last:
        hb = jax.nn.relu(hb)
    h_ref[...] = hb
    if not last:
        xln_ref[...] = jnp.dot(hb, wn_ref[...],
                               preferred_element_type=jnp.float32) + bn_ref[...]


_TC_PARAMS = pltpu.CompilerParams(vmem_limit_bytes=100 * 1024 * 1024)

_epi_tc = pl.pallas_call(
    functools.partial(_epi_body, False),
    out_shape=[
        jax.ShapeDtypeStruct((N, D), jnp.float32),
        jax.ShapeDtypeStruct((N, D), jnp.float32),
    ],
    compiler_params=_TC_PARAMS,
)

_epi_tc_last = pl.pallas_call(
    functools.partial(_epi_body, True),
    out_shape=[jax.ShapeDtypeStruct((N, D), jnp.float32)],
    compiler_params=_TC_PARAMS,
)


def kernel(x, edge_index, edge_attr, batch, atom_tab, bond_tab, W, b, root,
           gamma, beta):
    row = edge_index[0].astype(jnp.int32)
    col = edge_index[1].astype(jnp.int32)
    ea = edge_attr.astype(jnp.int32)
    aidx = ea[:, 0] * (BV * BV) + ea[:, 1] * BV + ea[:, 2]

    pad_e = EPAD - E
    row_p = jnp.concatenate([row, jnp.zeros((pad_e,), jnp.int32)])
    rowd_p = jnp.concatenate([row, jnp.full((pad_e,), N, jnp.int32)])
    col_p = jnp.concatenate([col, jnp.full((pad_e,), N, jnp.int32)])
    aidx_p = jnp.concatenate([aidx, jnp.zeros((pad_e,), jnp.int32)])
    rowd3 = rowd_p.reshape(NW, NCHUNK, CH)
    idx4 = jnp.stack([row_p.reshape(NCHT, CHL),
                      col_p.reshape(NCHT, CHL),
                      aidx_p.reshape(NCHT, CHL)], axis=1)

    zrow = jnp.zeros((RPT,), jnp.float32)

    degp = _deg_sc()(rowd3, zrow)

    b2 = b.reshape(LYR, 1, D)
    root2 = root.reshape(LYR, 1, D)
    gamma2 = gamma.reshape(LYR, 1, D)
    beta2 = beta.reshape(LYR, 1, D)

    h0, xl = _embed_tc(x.astype(jnp.int32), atom_tab, W[0], b2[0])
    dinv_col, deginv_col, ee = _misc_tc(degp, bond_tab)
    dinv_flat = dinv_col.reshape(NPAD)

    hs = [h0]
    for l in range(LYR):
        aggout = _layer_sc()(xl, ee[l], dinv_flat, idx4)
        sp = aggout[:, :N, :]
        if l < LYR - 1:
            h, xl = _epi_tc(sp, xl, dinv_col[:N], deginv_col, root2[l],
                            gamma2[l], beta2[l], W[l + 1], b2[l + 1])
        else:
            (h,) = _epi_tc_last(sp, xl, dinv_col[:N], deginv_col, root2[l],
                                gamma2[l], beta2[l])
        hs.append(h)

    return hs[-1], jnp.stack(hs, axis=0)
